# baseline (speedup 1.0000x reference)
import jax
import jax.numpy as jnp
from jax import lax
from jax.experimental import pallas as pl
from jax.experimental.pallas import tpu as pltpu
from jax.experimental.pallas import tpu_sc as plsc

N_NODES = 10000
N_EDGES = 320000
IN_DIM = 128
HIDDEN_DIM = 256

NUM_CORES = 2
NUM_SUBCORES = 16
NUM_WORKERS = NUM_CORES * NUM_SUBCORES
EDGES_PER_WORKER = N_EDGES // NUM_WORKERS
CHUNK = 128
NUM_CHUNKS = EDGES_PER_WORKER // CHUNK
TAIL = EDGES_PER_WORKER - NUM_CHUNKS * CHUNK
N_PAD = 10240
ROWS_PER_TILE = N_PAD // NUM_SUBCORES


def _sc_body(x_hbm, ei_hbm, part_hbm,
             s0_v, s1_v, d0_v, d1_v, rows0_v, rows1_v,
             st_v, dt_v, rt_v, acc_sh,
             gsem0, gsem1, ssem0, ssem1, isem0s, isem1s, isem0d, isem1d):
    c = lax.axis_index("c")
    s = lax.axis_index("s")
    wid = c * NUM_SUBCORES + s

    zv = jnp.zeros((16,), jnp.float32)

    def zrow(r, carry):
        def zcol(j, carry2):
            rows0_v[r, pl.ds(j * 16, 16)] = zv
            return carry2
        return lax.fori_loop(0, IN_DIM // 16, zcol, carry)
    lax.fori_loop(0, CHUNK, zrow, 0)

    row0 = s * ROWS_PER_TILE

    @pl.when(jnp.logical_and(c == 0, s < NUM_SUBCORES - 1))
    def _():
        pltpu.sync_copy(x_hbm.at[pl.ds(row0, ROWS_PER_TILE)],
                        acc_sh.at[pl.ds(row0, ROWS_PER_TILE)])

    @pl.when(jnp.logical_and(c == 0, s == NUM_SUBCORES - 1))
    def _():
        pltpu.sync_copy(x_hbm.at[pl.ds(9600, 400)], acc_sh.at[pl.ds(9600, 400)])

        def ztail(k, carry):
            pltpu.sync_copy(rows0_v.at[pl.ds(0, 80)],
                            acc_sh.at[pl.ds(N_NODES + k * 80, 80)])
            return carry
        lax.fori_loop(0, (N_PAD - N_NODES) // 80, ztail, 0)

    @pl.when(c == 1)
    def _():
        def zacc(k, carry):
            pltpu.sync_copy(rows0_v, acc_sh.at[pl.ds(row0 + k * CHUNK, CHUNK)])
            return carry
        lax.fori_loop(0, ROWS_PER_TILE // CHUNK, zacc, 0)

    ebase = wid * EDGES_PER_WORKER

    def src_at(a):
        return ei_hbm.at[pl.ds(ebase + a * CHUNK, CHUNK)]

    def dst_at(a):
        return ei_hbm.at[pl.ds(N_EDGES + ebase + a * CHUNK, CHUNK)]

    srcb = (s0_v, s1_v)
    dstb = (d0_v, d1_v)
    rows = (rows0_v, rows1_v)
    gsem = (gsem0, gsem1)
    ssem = (ssem0, ssem1)
    issem = (isem0s, isem1s)
    idsem = (isem0d, isem1d)

    def wait_src(a, q):
        pltpu.make_async_copy(src_at(a), srcb[q], issem[q]).wait()

    def wait_dst(a, q):
        pltpu.make_async_copy(dst_at(a), dstb[q], idsem[q]).wait()

    def wait_gather(q):
        pltpu.make_async_copy(x_hbm.at[srcb[q]], rows[q], gsem[q]).wait()

    def wait_scatter(q):
        pltpu.make_async_copy(rows[q], acc_sh.at[dstb[q]], ssem[q]).wait()

    def stage(a, x, gather_next, prefetch_src):
        y = 1 - x
        pltpu.make_async_copy(x_hbm.at[srcb[x]], rows[x], gsem[x]).wait()
        wait_dst(a, x)
        pltpu.async_copy(rows[x], acc_sh.at[dstb[x]], ssem[x], add=True)
        wait_scatter(y)
        if gather_next:
            wait_src(a + 1, y)
            pltpu.async_copy(x_hbm.at[srcb[y]], rows[y], gsem[y])
            pltpu.async_copy(dst_at(a + 1), dstb[y], idsem[y])
        if prefetch_src:
            pltpu.async_copy(src_at(a + 2), srcb[x], issem[x])

    pltpu.sync_copy(src_at(0), s0_v)
    plsc.subcore_barrier()
    pltpu.async_copy(x_hbm.at[s0_v], rows0_v, gsem0)
    pltpu.async_copy(dst_at(0), d0_v, isem0d)
    pltpu.async_copy(src_at(1), s1_v, isem1s)
    wait_gather(0)
    wait_dst(0, 0)
    pltpu.async_copy(rows0_v, acc_sh.at[d0_v], ssem0, add=True)
    wait_src(1, 1)
    pltpu.async_copy(x_hbm.at[s1_v], rows1_v, gsem1)
    pltpu.async_copy(dst_at(1), d1_v, isem1d)
    pltpu.async_copy(src_at(2), s0_v, isem0s)

    def pair_body(i, carry):
        a = 2 * i + 1
        stage(a, 1, True, True)
        stage(a + 1, 0, True, True)
        return carry
    lax.fori_loop(0, (NUM_CHUNKS - 3) // 2, pair_body, 0)
    stage(NUM_CHUNKS - 3, 1, True, True)
    stage(NUM_CHUNKS - 2, 0, True, False)
    stage(NUM_CHUNKS - 1, 1, False, False)
    wait_scatter(1)
    tbase = ebase + NUM_CHUNKS * CHUNK
    pltpu.sync_copy(ei_hbm.at[pl.ds(tbase, TAIL)], st_v)
    pltpu.sync_copy(ei_hbm.at[pl.ds(N_EDGES + tbase, TAIL)], dt_v)
    pltpu.async_copy(x_hbm.at[st_v], rt_v, gsem0).wait()
    pltpu.sync_copy(rt_v, acc_sh.at[dt_v], add=True)
    plsc.subcore_barrier()

    pltpu.sync_copy(acc_sh.at[pl.ds(row0, ROWS_PER_TILE)],
                    part_hbm.at[c, pl.ds(row0, ROWS_PER_TILE)])


@jax.jit
def _sc_scatter(x, ei):
    mesh = plsc.VectorSubcoreMesh(core_axis_name="c", subcore_axis_name="s")
    return pl.kernel(
        _sc_body,
        out_type=jax.ShapeDtypeStruct((NUM_CORES, N_PAD, IN_DIM), jnp.float32),
        mesh=mesh,
        scratch_types=[
            pltpu.VMEM((CHUNK,), jnp.int32),
            pltpu.VMEM((CHUNK,), jnp.int32),
            pltpu.VMEM((CHUNK,), jnp.int32),
            pltpu.VMEM((CHUNK,), jnp.int32),
            pltpu.VMEM((CHUNK, IN_DIM), jnp.float32),
            pltpu.VMEM((CHUNK, IN_DIM), jnp.float32),
            pltpu.VMEM((TAIL,), jnp.int32),
            pltpu.VMEM((TAIL,), jnp.int32),
            pltpu.VMEM((TAIL, IN_DIM), jnp.float32),
            pltpu.VMEM_SHARED((N_PAD, IN_DIM), jnp.float32),
            pltpu.SemaphoreType.DMA,
            pltpu.SemaphoreType.DMA,
            pltpu.SemaphoreType.DMA,
            pltpu.SemaphoreType.DMA,
            pltpu.SemaphoreType.DMA,
            pltpu.SemaphoreType.DMA,
            pltpu.SemaphoreType.DMA,
            pltpu.SemaphoreType.DMA,
        ],
    )(x, ei)


def _mlp_body(p_ref, w_ref, o_ref):
    h = p_ref[0] + p_ref[1]
    hw = lax.dot_general(h, w_ref[...], (((1,), (1,)), ((), ())),
                         preferred_element_type=jnp.float32)
    o_ref[...] = jnp.maximum(hw, 0.0)


@jax.jit
def _mlp(parts, w):
    blk = 1000
    grid = (N_NODES // blk,)
    return pl.pallas_call(
        _mlp_body,
        grid=grid,
        in_specs=[
            pl.BlockSpec((NUM_CORES, blk, IN_DIM), lambda i: (0, i, 0)),
            pl.BlockSpec((HIDDEN_DIM, IN_DIM), lambda i: (0, 0)),
        ],
        out_specs=pl.BlockSpec((blk, HIDDEN_DIM), lambda i: (i, 0)),
        out_shape=jax.ShapeDtypeStruct((N_NODES, HIDDEN_DIM), jnp.float32),
    )(parts, w)


def kernel(x, edge_index, W):
    ei = edge_index.astype(jnp.int32).reshape(2 * N_EDGES)
    parts = _sc_scatter(x, ei)
    return _mlp(parts, W)


# speedup vs baseline: 1.0190x; 1.0190x over previous
import jax
import jax.numpy as jnp
from jax import lax
from jax.experimental import pallas as pl
from jax.experimental.pallas import tpu as pltpu
from jax.experimental.pallas import tpu_sc as plsc

N_NODES = 10000
N_EDGES = 320000
IN_DIM = 128
HIDDEN_DIM = 256

NUM_CORES = 2
NUM_SUBCORES = 16
NUM_WORKERS = NUM_CORES * NUM_SUBCORES
EDGES_PER_WORKER = N_EDGES // NUM_WORKERS
CHUNK = 128
NUM_CHUNKS = EDGES_PER_WORKER // CHUNK
TAIL = EDGES_PER_WORKER - NUM_CHUNKS * CHUNK
N_PAD = 10240
ROWS_PER_TILE = N_PAD // NUM_SUBCORES


def _sc_body(x_hbm, ei_hbm, part_hbm,
             s0_v, s1_v, d0_v, d1_v, rows0_v, rows1_v,
             st_v, dt_v, rt_v, acc_sh,
             gsem0, gsem1, ssem0, ssem1, isem0s, isem1s, isem0d, isem1d):
    c = lax.axis_index("c")
    s = lax.axis_index("s")
    wid = c * NUM_SUBCORES + s

    zv = jnp.zeros((16,), jnp.float32)

    def zrow(r, carry):
        def zcol(j, carry2):
            rows0_v[r, pl.ds(j * 16, 16)] = zv
            return carry2
        return lax.fori_loop(0, IN_DIM // 16, zcol, carry)
    lax.fori_loop(0, CHUNK, zrow, 0)

    row0 = s * ROWS_PER_TILE

    def zacc(k, carry):
        pltpu.sync_copy(rows0_v, acc_sh.at[pl.ds(row0 + k * CHUNK, CHUNK)])
        return carry
    lax.fori_loop(0, ROWS_PER_TILE // CHUNK, zacc, 0)

    ebase = wid * EDGES_PER_WORKER

    def src_at(a):
        return ei_hbm.at[pl.ds(ebase + a * CHUNK, CHUNK)]

    def dst_at(a):
        return ei_hbm.at[pl.ds(N_EDGES + ebase + a * CHUNK, CHUNK)]

    srcb = (s0_v, s1_v)
    dstb = (d0_v, d1_v)
    rows = (rows0_v, rows1_v)
    gsem = (gsem0, gsem1)
    ssem = (ssem0, ssem1)
    issem = (isem0s, isem1s)
    idsem = (isem0d, isem1d)

    def wait_src(a, q):
        pltpu.make_async_copy(src_at(a), srcb[q], issem[q]).wait()

    def wait_dst(a, q):
        pltpu.make_async_copy(dst_at(a), dstb[q], idsem[q]).wait()

    def wait_gather(q):
        pltpu.make_async_copy(x_hbm.at[srcb[q]], rows[q], gsem[q]).wait()

    def wait_scatter(q):
        pltpu.make_async_copy(rows[q], acc_sh.at[dstb[q]], ssem[q]).wait()

    def stage(a, x, gather_next, prefetch_src):
        y = 1 - x
        pltpu.make_async_copy(x_hbm.at[srcb[x]], rows[x], gsem[x]).wait()
        wait_dst(a, x)
        pltpu.async_copy(rows[x], acc_sh.at[dstb[x]], ssem[x], add=True)
        wait_scatter(y)
        if gather_next:
            wait_src(a + 1, y)
            pltpu.async_copy(x_hbm.at[srcb[y]], rows[y], gsem[y])
            pltpu.async_copy(dst_at(a + 1), dstb[y], idsem[y])
        if prefetch_src:
            pltpu.async_copy(src_at(a + 2), srcb[x], issem[x])

    pltpu.sync_copy(src_at(0), s0_v)
    plsc.subcore_barrier()
    pltpu.async_copy(x_hbm.at[s0_v], rows0_v, gsem0)
    pltpu.async_copy(dst_at(0), d0_v, isem0d)
    pltpu.async_copy(src_at(1), s1_v, isem1s)
    wait_gather(0)
    wait_dst(0, 0)
    pltpu.async_copy(rows0_v, acc_sh.at[d0_v], ssem0, add=True)
    wait_src(1, 1)
    pltpu.async_copy(x_hbm.at[s1_v], rows1_v, gsem1)
    pltpu.async_copy(dst_at(1), d1_v, isem1d)
    pltpu.async_copy(src_at(2), s0_v, isem0s)

    def pair_body(i, carry):
        a = 2 * i + 1
        stage(a, 1, True, True)
        stage(a + 1, 0, True, True)
        return carry
    lax.fori_loop(0, (NUM_CHUNKS - 3) // 2, pair_body, 0)
    stage(NUM_CHUNKS - 3, 1, True, True)
    stage(NUM_CHUNKS - 2, 0, True, False)
    stage(NUM_CHUNKS - 1, 1, False, False)
    wait_scatter(1)
    tbase = ebase + NUM_CHUNKS * CHUNK
    pltpu.sync_copy(ei_hbm.at[pl.ds(tbase, TAIL)], st_v)
    pltpu.sync_copy(ei_hbm.at[pl.ds(N_EDGES + tbase, TAIL)], dt_v)
    pltpu.async_copy(x_hbm.at[st_v], rt_v, gsem0).wait()
    pltpu.sync_copy(rt_v, acc_sh.at[dt_v], add=True)
    plsc.subcore_barrier()

    pltpu.sync_copy(acc_sh.at[pl.ds(row0, ROWS_PER_TILE)],
                    part_hbm.at[c, pl.ds(row0, ROWS_PER_TILE)])


@jax.jit
def _sc_scatter(x, ei):
    mesh = plsc.VectorSubcoreMesh(core_axis_name="c", subcore_axis_name="s")
    return pl.kernel(
        _sc_body,
        out_type=jax.ShapeDtypeStruct((NUM_CORES, N_PAD, IN_DIM), jnp.float32),
        mesh=mesh,
        scratch_types=[
            pltpu.VMEM((CHUNK,), jnp.int32),
            pltpu.VMEM((CHUNK,), jnp.int32),
            pltpu.VMEM((CHUNK,), jnp.int32),
            pltpu.VMEM((CHUNK,), jnp.int32),
            pltpu.VMEM((CHUNK, IN_DIM), jnp.float32),
            pltpu.VMEM((CHUNK, IN_DIM), jnp.float32),
            pltpu.VMEM((TAIL,), jnp.int32),
            pltpu.VMEM((TAIL,), jnp.int32),
            pltpu.VMEM((TAIL, IN_DIM), jnp.float32),
            pltpu.VMEM_SHARED((N_PAD, IN_DIM), jnp.float32),
            pltpu.SemaphoreType.DMA,
            pltpu.SemaphoreType.DMA,
            pltpu.SemaphoreType.DMA,
            pltpu.SemaphoreType.DMA,
            pltpu.SemaphoreType.DMA,
            pltpu.SemaphoreType.DMA,
            pltpu.SemaphoreType.DMA,
            pltpu.SemaphoreType.DMA,
        ],
    )(x, ei)


def _mlp_body(x_ref, p_ref, w_ref, o_ref):
    h = x_ref[...] + p_ref[0] + p_ref[1]
    hw = lax.dot_general(h, w_ref[...], (((1,), (1,)), ((), ())),
                         preferred_element_type=jnp.float32)
    o_ref[...] = jnp.maximum(hw, 0.0)


@jax.jit
def _mlp(x, parts, w):
    blk = 1000
    grid = (N_NODES // blk,)
    return pl.pallas_call(
        _mlp_body,
        grid=grid,
        in_specs=[
            pl.BlockSpec((blk, IN_DIM), lambda i: (i, 0)),
            pl.BlockSpec((NUM_CORES, blk, IN_DIM), lambda i: (0, i, 0)),
            pl.BlockSpec((HIDDEN_DIM, IN_DIM), lambda i: (0, 0)),
        ],
        out_specs=pl.BlockSpec((blk, HIDDEN_DIM), lambda i: (i, 0)),
        out_shape=jax.ShapeDtypeStruct((N_NODES, HIDDEN_DIM), jnp.float32),
    )(x, parts, w)


def kernel(x, edge_index, W):
    ei = edge_index.astype(jnp.int32).reshape(2 * N_EDGES)
    parts = _sc_scatter(x, ei)
    return _mlp(x, parts, W)
